# Initial kernel scaffold; baseline (speedup 1.0000x reference)
import jax
import jax.numpy as jnp
from jax.experimental import pallas as pl


def kernel(sample, entity_embedding, relation_embedding):
    raise NotImplementedError("write your pallas kernel here")


# speedup vs baseline: 1.0434x; 1.0434x over previous
import functools

import jax
import jax.numpy as jnp
from jax import lax
from jax.experimental import pallas as pl
from jax.experimental.pallas import tpu as pltpu
from jax.experimental.pallas import tpu_sc as plsc

NENTITY = 1000000
NRELATION = 1000
HIDDEN = 128
GAMMA = 12.0
BATCH = 4096

NUM_CORES = 2
NUM_SUBCORES = 16
LANES = 16
NUM_WORKERS = NUM_CORES * NUM_SUBCORES
BPW = BATCH // NUM_WORKERS

_mesh = plsc.VectorSubcoreMesh(core_axis_name="c", subcore_axis_name="s")


@functools.partial(
    pl.kernel,
    mesh=_mesh,
    compiler_params=pltpu.CompilerParams(needs_layout_passes=False),
    out_type=jax.ShapeDtypeStruct((BATCH,), jnp.float32),
    scratch_types=[
        pltpu.VMEM((BPW,), jnp.int32),
        pltpu.VMEM((BPW,), jnp.int32),
        pltpu.VMEM((BPW,), jnp.int32),
        pltpu.VMEM((BPW, HIDDEN), jnp.float32),
        pltpu.VMEM((BPW, HIDDEN), jnp.float32),
        pltpu.VMEM((BPW, HIDDEN), jnp.float32),
        pltpu.VMEM((BPW,), jnp.float32),
        pltpu.SemaphoreType.DMA,
        pltpu.SemaphoreType.DMA,
        pltpu.SemaphoreType.DMA,
    ],
)
def _transe_sc(hidx_hbm, ridx_hbm, tidx_hbm, ent_hbm, rel_hbm, out_hbm,
               idx_h, idx_r, idx_t, rows_h, rows_r, rows_t, out_v,
               sem_h, sem_r, sem_t):
    wid = lax.axis_index("s") * NUM_CORES + lax.axis_index("c")
    base = wid * BPW

    pltpu.sync_copy(hidx_hbm.at[pl.ds(base, BPW)], idx_h)
    pltpu.sync_copy(ridx_hbm.at[pl.ds(base, BPW)], idx_r)
    pltpu.sync_copy(tidx_hbm.at[pl.ds(base, BPW)], idx_t)

    ch = pltpu.async_copy(ent_hbm.at[idx_h], rows_h, sem_h)
    cr = pltpu.async_copy(rel_hbm.at[idx_r], rows_r, sem_r)
    ct = pltpu.async_copy(ent_hbm.at[idx_t], rows_t, sem_t)
    ch.wait()
    cr.wait()
    ct.wait()

    lane_iota = lax.iota(jnp.int32, LANES)

    def group_body(g, carry):
        vec = jnp.zeros((LANES,), jnp.float32)
        for l in range(LANES):
            i = g * LANES + l
            acc = jnp.zeros((LANES,), jnp.float32)
            for j in range(HIDDEN // LANES):
                h = rows_h[i, pl.ds(j * LANES, LANES)]
                r = rows_r[i, pl.ds(j * LANES, LANES)]
                t = rows_t[i, pl.ds(j * LANES, LANES)]
                acc = acc + jnp.abs(h + r - t)
            s = jnp.sum(acc)
            vec = jnp.where(lane_iota == l, GAMMA - s, vec)
        out_v[pl.ds(pl.multiple_of(g * LANES, LANES), LANES)] = vec
        return carry

    lax.fori_loop(0, BPW // LANES, group_body, None)

    pltpu.sync_copy(out_v, out_hbm.at[pl.ds(base, BPW)])


def kernel(sample, entity_embedding, relation_embedding):
    head_idx = sample[:, 0]
    rel_idx = sample[:, 1]
    tail_idx = sample[:, 2]
    score = _transe_sc(head_idx, rel_idx, tail_idx,
                       entity_embedding, relation_embedding)
    return score.reshape(BATCH, 1)
